# baseline (speedup 1.0000x reference)
import functools

import jax
import jax.numpy as jnp
from jax import lax
from jax.experimental import pallas as pl
from jax.experimental.pallas import tpu as pltpu
from jax.experimental.pallas import tpu_sc as plsc

_NC, _NS, _L = 2, 16, 16
_NW = _NC * _NS
_CH = 64
_NBUF = 10


def _acc_rows(N):
    return ((N + _NS * 8 - 1) // (_NS * 8)) * _NS * 8


def _span_copy(sid, NA, copy_fn):
    rpt = NA // _NS
    copy_fn(sid * rpt, rpt)


def _make_deg_kernel(EP, NA):
    ept = EP // _NW
    nchunk = ept // _CH
    assert EP == ept * _NW and ept == nchunk * _CH
    mesh = plsc.VectorSubcoreMesh(core_axis_name="c", subcore_axis_name="s")

    @functools.partial(
        pl.kernel,
        out_type=jax.ShapeDtypeStruct((_NC, NA, _L), jnp.float32),
        mesh=mesh,
        compiler_params=pltpu.CompilerParams(use_tc_tiling_on_sc=False),
        scratch_types=[
            pltpu.VMEM_SHARED((NA, _L), jnp.float32),
            pltpu.VMEM((_CH, _L), jnp.float32),
            pltpu.VMEM((nchunk, _CH), jnp.int32),
            pltpu.SemaphoreType.DMA,
        ],
    )
    def deg_kernel(ei4_hbm, zero_hbm, out_hbm, acc, ones_v, idx_v, sem):
        cid = lax.axis_index("c")
        sid = lax.axis_index("s")

        def fill(i, carry):
            ones_v[i, :] = jnp.ones((_L,), jnp.float32)
            return carry

        lax.fori_loop(0, _CH, fill, None)
        pltpu.sync_copy(ei4_hbm.at[1, sid, pl.ds(cid * nchunk, nchunk)],
                        idx_v)
        _span_copy(sid, NA,
                   lambda r, n: pltpu.sync_copy(zero_hbm.at[pl.ds(r, n)],
                                                acc.at[pl.ds(r, n)]))
        plsc.subcore_barrier()

        def fire(i, carry):
            pltpu.async_copy(ones_v, acc.at[idx_v.at[i]], sem, add=True)
            return carry

        lax.fori_loop(0, nchunk, fire, None)

        def drain(i, carry):
            pltpu.make_async_copy(ones_v, acc.at[idx_v.at[0]], sem).wait()
            return carry

        lax.fori_loop(0, nchunk, drain, None)
        plsc.subcore_barrier()
        _span_copy(sid, NA,
                   lambda r, n: pltpu.sync_copy(acc.at[pl.ds(r, n)],
                                                out_hbm.at[cid, pl.ds(r, n)]))

    return deg_kernel


def _make_mp_kernel(EP, NA, D):
    dh = D // _NC
    ept = EP // _NS
    nchunk = ept // _CH
    assert EP == ept * _NS and ept == nchunk * _CH and D == dh * _NC
    assert nchunk % _NBUF == 0 and nchunk // _NBUF >= 2
    nmain = nchunk // _NBUF - 1
    mesh = plsc.VectorSubcoreMesh(core_axis_name="c", subcore_axis_name="s")

    @functools.partial(
        pl.kernel,
        out_type=jax.ShapeDtypeStruct((NA, D), jnp.float32),
        mesh=mesh,
        compiler_params=pltpu.CompilerParams(use_tc_tiling_on_sc=False),
        scratch_types=[
            pltpu.VMEM_SHARED((NA, dh), jnp.float32),
            pltpu.VMEM((_NBUF, _CH, dh), jnp.float32),
            pltpu.VMEM((nchunk, _CH), jnp.int32),
            pltpu.VMEM((nchunk, _CH), jnp.int32),
            [pltpu.SemaphoreType.DMA] * _NBUF,
            [pltpu.SemaphoreType.DMA] * _NBUF,
        ],
    )
    def mp_kernel(y3_hbm, ei4_hbm, zero_hbm, out_hbm,
                  acc, rows_v, src_v, dst_v, gsem, ssem):
        cid = lax.axis_index("c")
        sid = lax.axis_index("s")
        yh = y3_hbm.at[cid]

        pltpu.sync_copy(ei4_hbm.at[0, sid], src_v)
        pltpu.sync_copy(ei4_hbm.at[1, sid], dst_v)
        _span_copy(sid, NA,
                   lambda r, n: pltpu.sync_copy(zero_hbm.at[pl.ds(r, n)],
                                                acc.at[pl.ds(r, n)]))
        plsc.subcore_barrier()

        def fire_gather(g, b):
            pltpu.async_copy(yh.at[src_v.at[g]], rows_v.at[b], gsem[b])

        def wait_gather(b):
            pltpu.make_async_copy(yh.at[src_v.at[0]], rows_v.at[b],
                                  gsem[b]).wait()

        def fire_scatter(g, b):
            pltpu.async_copy(rows_v.at[b], acc.at[dst_v.at[g]], ssem[b],
                             add=True)

        def wait_scatter(b):
            pltpu.make_async_copy(rows_v.at[b], acc.at[dst_v.at[0]],
                                  ssem[b]).wait()

        for b in range(_NBUF):
            fire_gather(b, b)

        def ring(k, carry):
            g0 = k * _NBUF
            for b in range(_NBUF):
                wait_gather(b)
                fire_scatter(g0 + b, b)
            for b in range(_NBUF):
                wait_scatter(b)
                fire_gather(g0 + _NBUF + b, b)
            return carry

        lax.fori_loop(0, nmain, ring, None)

        g0 = nmain * _NBUF
        for b in range(_NBUF):
            wait_gather(b)
            fire_scatter(g0 + b, b)
        for b in range(_NBUF):
            wait_scatter(b)

        plsc.subcore_barrier()
        _span_copy(sid, NA,
                   lambda r, n: pltpu.sync_copy(
                       acc.at[pl.ds(r, n)],
                       out_hbm.at[pl.ds(r, n), pl.ds(cid * dh, dh)]))

    return mp_kernel


_RB = 2000


def _halves(pair_ref):
    return jnp.concatenate([pair_ref[0], pair_ref[1]], axis=1)


def _store_halves(pair_ref, x, dh):
    pair_ref[0] = x[:, :dh]
    pair_ref[1] = x[:, dh:]


def _dinv_block(degpair_ref):
    deg = degpair_ref[0, :, 0:1] + degpair_ref[1, :, 0:1] + 1.0
    return lax.rsqrt(deg)


def _tc1a_body(nf_ref, pw_ref, pb_ref, w1_ref, xw_ref):
    x0 = jnp.dot(nf_ref[...], pw_ref[...],
                 preferred_element_type=jnp.float32) + pb_ref[...]
    xw_ref[...] = jnp.dot(x0, w1_ref[...],
                          preferred_element_type=jnp.float32)


def _tc1b_body(degpair_ref, xw_ref, y1_ref):
    dinv = _dinv_block(degpair_ref)
    _store_halves(y1_ref, xw_ref[...] * dinv, xw_ref.shape[1] // _NC)


def _tc2_body(degpair_ref, acc_ref, y1_ref, b1_ref, w2_ref, y2_ref):
    dinv = _dinv_block(degpair_ref)
    h = jnp.maximum(dinv * (acc_ref[...] + _halves(y1_ref)) + b1_ref[...],
                    0.0)
    xw = jnp.dot(h, w2_ref[...], preferred_element_type=jnp.float32)
    _store_halves(y2_ref, xw * dinv, w2_ref.shape[1] // _NC)


def _tc3_body(degpair_ref, acc_ref, y2_ref, b2_ref, out_ref):
    dinv = _dinv_block(degpair_ref)
    out_ref[...] = jnp.maximum(
        dinv * (acc_ref[...] + _halves(y2_ref)) + b2_ref[...], 0.0)


def _row_spec(d):
    return pl.BlockSpec((_RB, d), lambda i: (i, 0))


def _pair_spec(d):
    return pl.BlockSpec((_NC, _RB, d), lambda i: (0, i, 0))


def _full_spec(r, c):
    return pl.BlockSpec((r, c), lambda i: (0, 0))


def kernel(node_features, edge_index, proj_W, proj_b, W1, b1, W2, b2):
    N, in_dim = node_features.shape
    E = edge_index.shape[1]
    proj_dim = proj_W.shape[1]
    hid = W1.shape[1]
    dh = hid // _NC
    NA = _acc_rows(N)

    unit = _NS * _CH * _NBUF * 2
    EP = ((E + unit - 1) // unit) * unit
    ei = edge_index.astype(jnp.int32)
    if EP > E:
        r = jnp.arange(EP - E, dtype=jnp.int32)
        pad = jnp.stack([r % N, N + r % (NA - N)])
        ei = jnp.concatenate([ei, pad], axis=1)
    nchunk_mp = EP // (_NS * _CH)
    ei4 = ei.reshape(2, _NS, nchunk_mp, _CH)
    zeros16 = jnp.zeros((NA, _L), jnp.float32)
    zeros_dh = jnp.zeros((NA, dh), jnp.float32)

    degpair = _make_deg_kernel(EP, NA)(ei4, zeros16)

    grid = (N // _RB,)
    xw1 = pl.pallas_call(
        _tc1a_body,
        grid=grid,
        in_specs=[_row_spec(in_dim), _full_spec(in_dim, proj_dim),
                  _full_spec(1, proj_dim), _full_spec(proj_dim, hid)],
        out_specs=_row_spec(hid),
        out_shape=jax.ShapeDtypeStruct((N, hid), jnp.float32),
    )(node_features, proj_W, proj_b.reshape(1, -1), W1)

    y1 = pl.pallas_call(
        _tc1b_body,
        grid=grid,
        in_specs=[_pair_spec(_L), _row_spec(hid)],
        out_specs=_pair_spec(dh),
        out_shape=jax.ShapeDtypeStruct((_NC, N, dh), jnp.float32),
    )(degpair, xw1)

    mp = _make_mp_kernel(EP, NA, hid)
    accp1 = mp(y1, ei4, zeros_dh)

    y2 = pl.pallas_call(
        _tc2_body,
        grid=grid,
        in_specs=[_pair_spec(_L), _row_spec(hid), _pair_spec(dh),
                  _full_spec(1, hid), _full_spec(hid, hid)],
        out_specs=_pair_spec(dh),
        out_shape=jax.ShapeDtypeStruct((_NC, N, dh), jnp.float32),
    )(degpair, accp1, y1, b1.reshape(1, -1), W2)

    accp2 = mp(y2, ei4, zeros_dh)

    out = pl.pallas_call(
        _tc3_body,
        grid=grid,
        in_specs=[_pair_spec(_L), _row_spec(hid), _pair_spec(dh),
                  _full_spec(1, hid)],
        out_specs=_row_spec(hid),
        out_shape=jax.ShapeDtypeStruct((N, hid), jnp.float32),
    )(degpair, accp2, y2, b2.reshape(1, -1))

    return out


# speedup vs baseline: 1.0093x; 1.0093x over previous
import functools

import jax
import jax.numpy as jnp
from jax import lax
from jax.experimental import pallas as pl
from jax.experimental.pallas import tpu as pltpu
from jax.experimental.pallas import tpu_sc as plsc

_NC, _NS, _L = 2, 16, 16
_NW = _NC * _NS
_CH = 64
_NBUF = 8


def _acc_rows(N):
    return ((N + _NS * 8 - 1) // (_NS * 8)) * _NS * 8


def _span_copy(sid, NA, copy_fn):
    rpt = NA // _NS
    copy_fn(sid * rpt, rpt)


def _make_deg_kernel(EP, NA):
    ept = EP // _NW
    nchunk = ept // _CH
    assert EP == ept * _NW and ept == nchunk * _CH
    mesh = plsc.VectorSubcoreMesh(core_axis_name="c", subcore_axis_name="s")

    @functools.partial(
        pl.kernel,
        out_type=jax.ShapeDtypeStruct((_NC, NA, _L), jnp.float32),
        mesh=mesh,
        compiler_params=pltpu.CompilerParams(use_tc_tiling_on_sc=False),
        scratch_types=[
            pltpu.VMEM_SHARED((NA, _L), jnp.float32),
            pltpu.VMEM((_CH, _L), jnp.float32),
            pltpu.VMEM((nchunk, _CH), jnp.int32),
            pltpu.SemaphoreType.DMA,
        ],
    )
    def deg_kernel(ei4_hbm, zero_hbm, out_hbm, acc, ones_v, idx_v, sem):
        cid = lax.axis_index("c")
        sid = lax.axis_index("s")

        def fill(i, carry):
            ones_v[i, :] = jnp.ones((_L,), jnp.float32)
            return carry

        lax.fori_loop(0, _CH, fill, None)
        pltpu.sync_copy(ei4_hbm.at[1, sid, pl.ds(cid * nchunk, nchunk)],
                        idx_v)
        _span_copy(sid, NA,
                   lambda r, n: pltpu.sync_copy(zero_hbm.at[pl.ds(r, n)],
                                                acc.at[pl.ds(r, n)]))
        plsc.subcore_barrier()

        def fire(i, carry):
            pltpu.async_copy(ones_v, acc.at[idx_v.at[i]], sem, add=True)
            return carry

        lax.fori_loop(0, nchunk, fire, None)

        def drain(i, carry):
            pltpu.make_async_copy(ones_v, acc.at[idx_v.at[0]], sem).wait()
            return carry

        lax.fori_loop(0, nchunk, drain, None)
        plsc.subcore_barrier()
        _span_copy(sid, NA,
                   lambda r, n: pltpu.sync_copy(acc.at[pl.ds(r, n)],
                                                out_hbm.at[cid, pl.ds(r, n)]))

    return deg_kernel


def _make_mp_kernel(EP, NA, D):
    dh = D // _NC
    ept = EP // _NS
    nchunk = ept // _CH
    assert EP == ept * _NS and ept == nchunk * _CH and D == dh * _NC
    assert nchunk % _NBUF == 0 and nchunk // _NBUF >= 2
    nmain = nchunk // _NBUF - 1
    mesh = plsc.VectorSubcoreMesh(core_axis_name="c", subcore_axis_name="s")

    @functools.partial(
        pl.kernel,
        out_type=jax.ShapeDtypeStruct((NA, D), jnp.float32),
        mesh=mesh,
        compiler_params=pltpu.CompilerParams(use_tc_tiling_on_sc=False),
        scratch_types=[
            pltpu.VMEM_SHARED((NA, dh), jnp.float32),
            pltpu.VMEM((_NBUF, _CH, dh), jnp.float32),
            pltpu.VMEM((nchunk, _CH), jnp.int32),
            pltpu.VMEM((nchunk, _CH), jnp.int32),
            [pltpu.SemaphoreType.DMA] * _NBUF,
            [pltpu.SemaphoreType.DMA] * _NBUF,
        ],
    )
    def mp_kernel(y3_hbm, ei4_hbm, zero_hbm, out_hbm,
                  acc, rows_v, src_v, dst_v, gsem, ssem):
        cid = lax.axis_index("c")
        sid = lax.axis_index("s")
        yh = y3_hbm.at[cid]

        pltpu.sync_copy(ei4_hbm.at[0, sid], src_v)
        pltpu.sync_copy(ei4_hbm.at[1, sid], dst_v)
        _span_copy(sid, NA,
                   lambda r, n: pltpu.sync_copy(zero_hbm.at[pl.ds(r, n)],
                                                acc.at[pl.ds(r, n)]))
        plsc.subcore_barrier()

        def fire_gather(g, b):
            pltpu.async_copy(yh.at[src_v.at[g]], rows_v.at[b], gsem[b])

        def wait_gather(b):
            pltpu.make_async_copy(yh.at[src_v.at[0]], rows_v.at[b],
                                  gsem[b]).wait()

        def fire_scatter(g, b):
            pltpu.async_copy(rows_v.at[b], acc.at[dst_v.at[g]], ssem[b],
                             add=True)

        def wait_scatter(b):
            pltpu.make_async_copy(rows_v.at[b], acc.at[dst_v.at[0]],
                                  ssem[b]).wait()

        for b in range(_NBUF):
            fire_gather(b, b)

        def ring(k, carry):
            g0 = k * _NBUF
            for b in range(_NBUF):
                wait_gather(b)
                fire_scatter(g0 + b, b)
            for b in range(_NBUF):
                wait_scatter(b)
                fire_gather(g0 + _NBUF + b, b)
            return carry

        lax.fori_loop(0, nmain, ring, None)

        g0 = nmain * _NBUF
        for b in range(_NBUF):
            wait_gather(b)
            fire_scatter(g0 + b, b)
        for b in range(_NBUF):
            wait_scatter(b)

        plsc.subcore_barrier()
        _span_copy(sid, NA,
                   lambda r, n: pltpu.sync_copy(
                       acc.at[pl.ds(r, n)],
                       out_hbm.at[pl.ds(r, n), pl.ds(cid * dh, dh)]))

    return mp_kernel


_RB = 2000


def _halves(pair_ref):
    return jnp.concatenate([pair_ref[0], pair_ref[1]], axis=1)


def _store_halves(pair_ref, x, dh):
    pair_ref[0] = x[:, :dh]
    pair_ref[1] = x[:, dh:]


def _dinv_block(degpair_ref):
    deg = degpair_ref[0, :, 0:1] + degpair_ref[1, :, 0:1] + 1.0
    return lax.rsqrt(deg)


def _tc1a_body(nf_ref, pw_ref, pb_ref, w1_ref, xw_ref):
    x0 = jnp.dot(nf_ref[...], pw_ref[...],
                 preferred_element_type=jnp.float32) + pb_ref[...]
    xw_ref[...] = jnp.dot(x0, w1_ref[...],
                          preferred_element_type=jnp.float32)


def _tc1b_body(degpair_ref, xw_ref, y1_ref):
    dinv = _dinv_block(degpair_ref)
    _store_halves(y1_ref, xw_ref[...] * dinv, xw_ref.shape[1] // _NC)


def _tc2_body(degpair_ref, acc_ref, y1_ref, b1_ref, w2_ref, y2_ref):
    dinv = _dinv_block(degpair_ref)
    h = jnp.maximum(dinv * (acc_ref[...] + _halves(y1_ref)) + b1_ref[...],
                    0.0)
    xw = jnp.dot(h, w2_ref[...], preferred_element_type=jnp.float32)
    _store_halves(y2_ref, xw * dinv, w2_ref.shape[1] // _NC)


def _tc3_body(degpair_ref, acc_ref, y2_ref, b2_ref, out_ref):
    dinv = _dinv_block(degpair_ref)
    out_ref[...] = jnp.maximum(
        dinv * (acc_ref[...] + _halves(y2_ref)) + b2_ref[...], 0.0)


def _row_spec(d):
    return pl.BlockSpec((_RB, d), lambda i: (i, 0))


def _pair_spec(d):
    return pl.BlockSpec((_NC, _RB, d), lambda i: (0, i, 0))


def _full_spec(r, c):
    return pl.BlockSpec((r, c), lambda i: (0, 0))


def kernel(node_features, edge_index, proj_W, proj_b, W1, b1, W2, b2):
    N, in_dim = node_features.shape
    E = edge_index.shape[1]
    proj_dim = proj_W.shape[1]
    hid = W1.shape[1]
    dh = hid // _NC
    NA = _acc_rows(N)

    unit = _NS * _CH * _NBUF * 2
    EP = ((E + unit - 1) // unit) * unit
    ei = edge_index.astype(jnp.int32)
    if EP > E:
        r = jnp.arange(EP - E, dtype=jnp.int32)
        pad = jnp.stack([r % N, N + r % (NA - N)])
        ei = jnp.concatenate([ei, pad], axis=1)
    nchunk_mp = EP // (_NS * _CH)
    ei4 = ei.reshape(2, _NS, nchunk_mp, _CH)
    zeros16 = jnp.zeros((NA, _L), jnp.float32)
    zeros_dh = jnp.zeros((NA, dh), jnp.float32)

    degpair = _make_deg_kernel(EP, NA)(ei4, zeros16)

    grid = (N // _RB,)
    xw1 = pl.pallas_call(
        _tc1a_body,
        grid=grid,
        in_specs=[_row_spec(in_dim), _full_spec(in_dim, proj_dim),
                  _full_spec(1, proj_dim), _full_spec(proj_dim, hid)],
        out_specs=_row_spec(hid),
        out_shape=jax.ShapeDtypeStruct((N, hid), jnp.float32),
    )(node_features, proj_W, proj_b.reshape(1, -1), W1)

    y1 = pl.pallas_call(
        _tc1b_body,
        grid=grid,
        in_specs=[_pair_spec(_L), _row_spec(hid)],
        out_specs=_pair_spec(dh),
        out_shape=jax.ShapeDtypeStruct((_NC, N, dh), jnp.float32),
    )(degpair, xw1)

    mp = _make_mp_kernel(EP, NA, hid)
    accp1 = mp(y1, ei4, zeros_dh)

    y2 = pl.pallas_call(
        _tc2_body,
        grid=grid,
        in_specs=[_pair_spec(_L), _row_spec(hid), _pair_spec(dh),
                  _full_spec(1, hid), _full_spec(hid, hid)],
        out_specs=_pair_spec(dh),
        out_shape=jax.ShapeDtypeStruct((_NC, N, dh), jnp.float32),
    )(degpair, accp1, y1, b1.reshape(1, -1), W2)

    accp2 = mp(y2, ei4, zeros_dh)

    out = pl.pallas_call(
        _tc3_body,
        grid=grid,
        in_specs=[_pair_spec(_L), _row_spec(hid), _pair_spec(dh),
                  _full_spec(1, hid)],
        out_specs=_row_spec(hid),
        out_shape=jax.ShapeDtypeStruct((N, hid), jnp.float32),
    )(degpair, accp2, y2, b2.reshape(1, -1))

    return out
